# baseline (speedup 1.0000x reference)
import functools

import jax
import jax.numpy as jnp
from jax import lax
from jax.experimental import pallas as pl
from jax.experimental.pallas import tpu as pltpu
from jax.experimental.pallas import tpu_sc as plsc

B = 16384
D = 64
LANES = 16
NC = 2
NS = 16
NW = NC * NS
BPW = B // NW
CHUNK = 128
NCHUNK = BPW // CHUNK
GROUPS = BPW // LANES


def _body(uid_hbm, iid_hbm, ut_hbm, it_hbm, out_hbm,
          uidx, iidx, urows, irows, outv, tpose, sem):
    wid = lax.axis_index("s") * NC + lax.axis_index("c")
    base = wid * BPW

    for j in range(NCHUNK):
        pltpu.sync_copy(uid_hbm.at[pl.ds(base + j * CHUNK, CHUNK)], uidx.at[j])
        pltpu.sync_copy(iid_hbm.at[pl.ds(base + j * CHUNK, CHUNK)], iidx.at[j])

    copies = []
    for j in range(NCHUNK):
        copies.append(pltpu.async_copy(
            ut_hbm.at[uidx.at[j]], urows.at[pl.ds(j * CHUNK, CHUNK)], sem))
        copies.append(pltpu.async_copy(
            it_hbm.at[iidx.at[j]], irows.at[pl.ds(j * CHUNK, CHUNK)], sem))
    for c in copies:
        c.wait()

    lane = lax.iota(jnp.int32, LANES)

    def group(g, carry):
        for r in range(LANES):
            row = g * LANES + r
            s = jnp.zeros((LANES,), jnp.float32)
            for k in range(D // LANES):
                uu = urows[row, pl.ds(k * LANES, LANES)]
                vv = irows[row, pl.ds(k * LANES, LANES)]
                s = s + uu * vv
            plsc.store_scatter(tpose, [lane * LANES + r], s)
        acc = jnp.zeros((LANES,), jnp.float32)
        for l in range(LANES):
            acc = acc + tpose[pl.ds(l * LANES, LANES)]
        outv[pl.ds(g * LANES, LANES)] = acc
        return carry

    lax.fori_loop(0, GROUPS, group, 0)

    pltpu.sync_copy(outv, out_hbm.at[pl.ds(base, BPW)])


def kernel(user_ids, item_ids, user_table, item_table):
    mesh = plsc.VectorSubcoreMesh(core_axis_name="c", subcore_axis_name="s")
    run = functools.partial(
        pl.kernel,
        mesh=mesh,
        compiler_params=pltpu.CompilerParams(
            needs_layout_passes=False, use_tc_tiling_on_sc=False),
        out_type=jax.ShapeDtypeStruct((B,), jnp.float32),
        scratch_types=[
            pltpu.VMEM((NCHUNK, CHUNK), jnp.int32),
            pltpu.VMEM((NCHUNK, CHUNK), jnp.int32),
            pltpu.VMEM((BPW, D), jnp.float32),
            pltpu.VMEM((BPW, D), jnp.float32),
            pltpu.VMEM((BPW,), jnp.float32),
            pltpu.VMEM((LANES * LANES,), jnp.float32),
            pltpu.SemaphoreType.DMA,
        ],
    )(_body)
    return run(user_ids.astype(jnp.int32), item_ids.astype(jnp.int32),
               user_table, item_table)
